# baseline (speedup 1.0000x reference)
import jax
import jax.numpy as jnp
from jax import lax
from jax.experimental import pallas as pl
from jax.experimental.pallas import tpu as pltpu
from jax.experimental.pallas import tpu_sc as plsc

N = 10000
D = 128
NC = 2
NS = 16
NW = NC * NS
C = 72
NCHUNK = 144
EPW = NCHUNK * C
E_PAD = NW * EPW
N_ACC = 10240
ROWS_PER_TILE = N_ACC // NS
VECS = D // 16


def _sc_aggregate_body(h_hbm, src_hbm, dst_hbm, w_hbm, out_hbm,
                       ed, wf, buf0, buf1, buf2, buf3,
                       esem0, esem1, esem2, esem3, esem4, esem5,
                       rsem0, rsem1, rsem2, rsem3,
                       ssem0, ssem1, ssem2, ssem3, acc):
    c = lax.axis_index("c")
    s = lax.axis_index("s")
    wid = c * NS + s
    ebase = wid * EPW

    esems = (esem0, esem1, esem2, esem3, esem4, esem5)
    bufs = (buf0, buf1, buf2, buf3)
    rsems = (rsem0, rsem1, rsem2, rsem3)
    ssems = (ssem0, ssem1, ssem2, ssem3)


    def _ed_copies(j, es):
        off = ebase + j * C
        return (
            pltpu.make_async_copy(src_hbm.at[pl.ds(off, C)], ed.at[es, 0],
                                  esems[es]),
            pltpu.make_async_copy(dst_hbm.at[pl.ds(off, C)], ed.at[es, 1],
                                  esems[es]),
            pltpu.make_async_copy(w_hbm.at[pl.ds(off * 16, C * 16)],
                                  wf.at[es], esems[es]),
        )

    def _ed_issue(j, es):
        for cp in _ed_copies(j, es):
            cp.start()

    def _ed_wait(j, es):
        for cp in _ed_copies(j, es):
            cp.wait()

    def _gather(j, bs, es):
        return pltpu.make_async_copy(h_hbm.at[ed.at[es, 0]], bufs[bs],
                                     rsems[bs])

    def _scatter(bs, es):
        return pltpu.make_async_copy(bufs[bs], acc.at[ed.at[es, 1]],
                                     ssems[bs])

    def _zero_row(e, _):
        for v in range(VECS):
            buf0[e, pl.ds(v * 16, 16)] = jnp.zeros((16,), jnp.float32)
        return _
    lax.fori_loop(0, C, _zero_row, None)
    full = ROWS_PER_TILE // C
    for i in range(full):
        pltpu.sync_copy(buf0, acc.at[pl.ds(s * ROWS_PER_TILE + i * C, C)])
    rem = ROWS_PER_TILE - full * C
    if rem:
        pltpu.sync_copy(buf0.at[pl.ds(0, rem)],
                        acc.at[pl.ds(s * ROWS_PER_TILE + full * C, rem)])

    for j in range(4):
        _ed_issue(j, j)
    _ed_wait(0, 0)
    _gather(0, 0, 0).start()
    _ed_wait(1, 1)
    _gather(1, 1, 1).start()

    plsc.subcore_barrier()

    def _scale(buf, es):
        def _edge(e, _):
            w16 = wf[es, pl.ds(e * 16, 16)]
            for v in range(VECS):
                sl = (e, pl.ds(v * 16, 16))
                buf[sl] = buf[sl] * w16
            return _
        lax.fori_loop(0, C, _edge, None)

    def _chunk(j, bs, es):

        @pl.when(j >= 2)
        def _():
            _scatter((bs + 2) % 4, (es + 4) % 6).wait()

        @pl.when(j + 2 < NCHUNK)
        def _():
            _ed_wait(j + 2, (es + 2) % 6)
            _gather(j + 2, (bs + 2) % 4, (es + 2) % 6).start()

        _gather(j, bs, es).wait()
        _scale(bufs[bs], es)
        pltpu.async_copy(bufs[bs], acc.at[ed.at[es, 1]], ssems[bs],
                         add=True)

        @pl.when(j + 4 < NCHUNK)
        def _():
            _ed_issue(j + 4, (es + 4) % 6)

    def _twelve(q, _):
        j0 = q * 12
        for b in range(12):
            _chunk(j0 + b, b % 4, b % 6)
        return _
    lax.fori_loop(0, NCHUNK // 12, _twelve, None)
    _scatter((NCHUNK - 2) % 4, (NCHUNK - 2) % 6).wait()
    _scatter((NCHUNK - 1) % 4, (NCHUNK - 1) % 6).wait()

    plsc.subcore_barrier()
    pltpu.sync_copy(acc.at[pl.ds(s * ROWS_PER_TILE, ROWS_PER_TILE)],
                    out_hbm.at[c, pl.ds(s * ROWS_PER_TILE, ROWS_PER_TILE)])


_sc_aggregate = pl.kernel(
    _sc_aggregate_body,
    out_type=jax.ShapeDtypeStruct((NC, N_ACC, D), jnp.float32),
    mesh=plsc.VectorSubcoreMesh(core_axis_name="c", subcore_axis_name="s"),
    scratch_types=[
        pltpu.VMEM((6, 3, C), jnp.int32),
        pltpu.VMEM((6, C * 16), jnp.float32),
        pltpu.VMEM((C, D), jnp.float32),
        pltpu.VMEM((C, D), jnp.float32),
        pltpu.VMEM((C, D), jnp.float32),
        pltpu.VMEM((C, D), jnp.float32),
        pltpu.SemaphoreType.DMA,
        pltpu.SemaphoreType.DMA,
        pltpu.SemaphoreType.DMA,
        pltpu.SemaphoreType.DMA,
        pltpu.SemaphoreType.DMA,
        pltpu.SemaphoreType.DMA,
        pltpu.SemaphoreType.DMA,
        pltpu.SemaphoreType.DMA,
        pltpu.SemaphoreType.DMA,
        pltpu.SemaphoreType.DMA,
        pltpu.SemaphoreType.DMA,
        pltpu.SemaphoreType.DMA,
        pltpu.SemaphoreType.DMA,
        pltpu.SemaphoreType.DMA,
        pltpu.VMEM_SHARED((N_ACC, D), jnp.float32),
    ],
)


_ROWS = 2000


def _matmul_body(x_ref, w_ref, o_ref):
    o_ref[...] = jnp.dot(x_ref[...], w_ref[...],
                         preferred_element_type=jnp.float32)


@jax.jit
def _tc_matmul(x, w):
    return pl.pallas_call(
        _matmul_body,
        grid=(N // _ROWS,),
        in_specs=[
            pl.BlockSpec((_ROWS, D), lambda i: (i, 0)),
            pl.BlockSpec((D, D), lambda i: (0, 0)),
        ],
        out_specs=pl.BlockSpec((_ROWS, D), lambda i: (i, 0)),
        out_shape=jax.ShapeDtypeStruct((N, D), jnp.float32),
    )(x, w)


def _mid_body(p0_ref, p1_ref, b_ref, w_ref, o_ref):
    h1 = jnp.tanh(p0_ref[...] + p1_ref[...] + b_ref[...])
    o_ref[...] = jnp.dot(h1, w_ref[...], preferred_element_type=jnp.float32)


@jax.jit
def _tc_mid(p0, p1, b, w):
    return pl.pallas_call(
        _mid_body,
        grid=(N // _ROWS,),
        in_specs=[
            pl.BlockSpec((_ROWS, D), lambda i: (i, 0)),
            pl.BlockSpec((_ROWS, D), lambda i: (i, 0)),
            pl.BlockSpec((1, D), lambda i: (0, 0)),
            pl.BlockSpec((D, D), lambda i: (0, 0)),
        ],
        out_specs=pl.BlockSpec((_ROWS, D), lambda i: (i, 0)),
        out_shape=jax.ShapeDtypeStruct((N, D), jnp.float32),
    )(p0, p1, b, w)


def _final_body(p0_ref, p1_ref, b_ref, o_ref):
    o_ref[...] = jnp.tanh(p0_ref[...] + p1_ref[...] + b_ref[...])


@jax.jit
def _tc_final(p0, p1, b):
    return pl.pallas_call(
        _final_body,
        grid=(N // _ROWS,),
        in_specs=[
            pl.BlockSpec((_ROWS, D), lambda i: (i, 0)),
            pl.BlockSpec((_ROWS, D), lambda i: (i, 0)),
            pl.BlockSpec((1, D), lambda i: (0, 0)),
        ],
        out_specs=pl.BlockSpec((_ROWS, D), lambda i: (i, 0)),
        out_shape=jax.ShapeDtypeStruct((N, D), jnp.float32),
    )(p0, p1, b)


@jax.jit
def kernel(x, edge_index, edge_weight, W1, b1, W2, b2):
    pad = E_PAD - edge_index.shape[1]
    pad_idx = (jnp.arange(pad, dtype=jnp.int32) % N)
    src = jnp.concatenate([edge_index[0].astype(jnp.int32), pad_idx])
    dst = jnp.concatenate([edge_index[1].astype(jnp.int32), pad_idx])
    w1d = jnp.concatenate([edge_weight.astype(jnp.float32),
                           jnp.zeros((pad,), jnp.float32)])
    w = jnp.broadcast_to(w1d[:, None], (E_PAD, 16)).reshape(-1)
    b1r = b1.reshape(1, D)
    b2r = b2.reshape(1, D)

    h = _tc_matmul(x, W1)
    p = _sc_aggregate(h, src, dst, w)
    h2 = _tc_mid(p[0, :N], p[1, :N], b1r, W2)
    p2 = _sc_aggregate(h2, src, dst, w)
    return _tc_final(p2[0, :N], p2[1, :N], b2r)


# speedup vs baseline: 1.7255x; 1.7255x over previous
import jax
import jax.numpy as jnp
from jax import lax
from jax.experimental import pallas as pl
from jax.experimental.pallas import tpu as pltpu
from jax.experimental.pallas import tpu_sc as plsc

N = 10000
D = 128
NC = 2
NS = 16
NW = NC * NS
C = 88
NCHUNK = 120
EPW = NCHUNK * C
E_PAD = NW * EPW
N_ACC = 10240
ROWS_PER_TILE = N_ACC // NS
VECS = D // 16


def _sc_aggregate_body(h_hbm, src_hbm, dst_hbm, w_hbm, out_hbm,
                       ed, wf, buf0, buf1, buf2, buf3,
                       esem0, esem1, esem2, esem3, esem4, esem5,
                       rsem0, rsem1, rsem2, rsem3,
                       ssem0, ssem1, ssem2, ssem3, acc):
    c = lax.axis_index("c")
    s = lax.axis_index("s")
    wid = c * NS + s
    ebase = wid * EPW

    esems = (esem0, esem1, esem2, esem3, esem4, esem5)
    bufs = (buf0, buf1, buf2, buf3)
    rsems = (rsem0, rsem1, rsem2, rsem3)
    ssems = (ssem0, ssem1, ssem2, ssem3)


    def _ed_copies(j, es):
        off = ebase + j * C
        return (
            pltpu.make_async_copy(src_hbm.at[pl.ds(off, C)], ed.at[es, 0],
                                  esems[es]),
            pltpu.make_async_copy(dst_hbm.at[pl.ds(off, C)], ed.at[es, 1],
                                  esems[es]),
            pltpu.make_async_copy(w_hbm.at[pl.ds(off, C)],
                                  wf.at[es, pl.ds(0, C)], esems[es]),
        )

    def _ed_issue(j, es):
        for cp in _ed_copies(j, es):
            cp.start()

    def _ed_wait(j, es):
        for cp in _ed_copies(j, es):
            cp.wait()

    def _gather(j, bs, es):
        return pltpu.make_async_copy(h_hbm.at[ed.at[es, 0]], bufs[bs],
                                     rsems[bs])

    def _scatter(bs, es):
        return pltpu.make_async_copy(bufs[bs], acc.at[ed.at[es, 1]],
                                     ssems[bs])

    def _zero_row(e, _):
        for v in range(VECS):
            buf0[e, pl.ds(v * 16, 16)] = jnp.zeros((16,), jnp.float32)
        return _
    lax.fori_loop(0, C, _zero_row, None)
    full = ROWS_PER_TILE // C
    for i in range(full):
        pltpu.sync_copy(buf0, acc.at[pl.ds(s * ROWS_PER_TILE + i * C, C)])
    rem = ROWS_PER_TILE - full * C
    if rem:
        pltpu.sync_copy(buf0.at[pl.ds(0, rem)],
                        acc.at[pl.ds(s * ROWS_PER_TILE + full * C, rem)])

    for j in range(4):
        _ed_issue(j, j)
    _ed_wait(0, 0)
    _gather(0, 0, 0).start()
    _ed_wait(1, 1)
    _gather(1, 1, 1).start()

    plsc.subcore_barrier()

    def _scale(buf, es):
        def _grp(g, _):
            wv = wf[es, pl.ds(g * 16, 16)]
            for k in range(16):
                w = wv[k]
                for v in range(VECS):
                    sl = (g * 16 + k, pl.ds(v * 16, 16))
                    buf[sl] = buf[sl] * w
            return _
        lax.fori_loop(0, C // 16, _grp, None)
        g = C // 16
        for k in range(C - g * 16):
            w = wf[es, pl.ds(g * 16, 16)][k]
            for v in range(VECS):
                sl = (g * 16 + k, pl.ds(v * 16, 16))
                buf[sl] = buf[sl] * w

    def _chunk(j, bs, es):

        @pl.when(j >= 2)
        def _():
            _scatter((bs + 2) % 4, (es + 4) % 6).wait()

        @pl.when(j + 2 < NCHUNK)
        def _():
            _ed_wait(j + 2, (es + 2) % 6)
            _gather(j + 2, (bs + 2) % 4, (es + 2) % 6).start()

        _gather(j, bs, es).wait()
        _scale(bufs[bs], es)
        pltpu.async_copy(bufs[bs], acc.at[ed.at[es, 1]], ssems[bs],
                         add=True)

        @pl.when(j + 4 < NCHUNK)
        def _():
            _ed_issue(j + 4, (es + 4) % 6)

    def _twelve(q, _):
        j0 = q * 12
        for b in range(12):
            _chunk(j0 + b, b % 4, b % 6)
        return _
    lax.fori_loop(0, NCHUNK // 12, _twelve, None)
    _scatter((NCHUNK - 2) % 4, (NCHUNK - 2) % 6).wait()
    _scatter((NCHUNK - 1) % 4, (NCHUNK - 1) % 6).wait()

    plsc.subcore_barrier()
    pltpu.sync_copy(acc.at[pl.ds(s * ROWS_PER_TILE, ROWS_PER_TILE)],
                    out_hbm.at[c, pl.ds(s * ROWS_PER_TILE, ROWS_PER_TILE)])


_sc_aggregate = pl.kernel(
    _sc_aggregate_body,
    out_type=jax.ShapeDtypeStruct((NC, N_ACC, D), jnp.float32),
    mesh=plsc.VectorSubcoreMesh(core_axis_name="c", subcore_axis_name="s"),
    scratch_types=[
        pltpu.VMEM((6, 3, C), jnp.int32),
        pltpu.VMEM((6, 128), jnp.float32),
        pltpu.VMEM((C, D), jnp.float32),
        pltpu.VMEM((C, D), jnp.float32),
        pltpu.VMEM((C, D), jnp.float32),
        pltpu.VMEM((C, D), jnp.float32),
        pltpu.SemaphoreType.DMA,
        pltpu.SemaphoreType.DMA,
        pltpu.SemaphoreType.DMA,
        pltpu.SemaphoreType.DMA,
        pltpu.SemaphoreType.DMA,
        pltpu.SemaphoreType.DMA,
        pltpu.SemaphoreType.DMA,
        pltpu.SemaphoreType.DMA,
        pltpu.SemaphoreType.DMA,
        pltpu.SemaphoreType.DMA,
        pltpu.SemaphoreType.DMA,
        pltpu.SemaphoreType.DMA,
        pltpu.SemaphoreType.DMA,
        pltpu.SemaphoreType.DMA,
        pltpu.VMEM_SHARED((N_ACC, D), jnp.float32),
    ],
)


_ROWS = 2000


def _matmul_body(x_ref, w_ref, o_ref):
    o_ref[...] = jnp.dot(x_ref[...], w_ref[...],
                         preferred_element_type=jnp.float32)


@jax.jit
def _tc_matmul(x, w):
    return pl.pallas_call(
        _matmul_body,
        grid=(N // _ROWS,),
        in_specs=[
            pl.BlockSpec((_ROWS, D), lambda i: (i, 0)),
            pl.BlockSpec((D, D), lambda i: (0, 0)),
        ],
        out_specs=pl.BlockSpec((_ROWS, D), lambda i: (i, 0)),
        out_shape=jax.ShapeDtypeStruct((N, D), jnp.float32),
    )(x, w)


def _mid_body(p0_ref, p1_ref, b_ref, w_ref, o_ref):
    h1 = jnp.tanh(p0_ref[...] + p1_ref[...] + b_ref[...])
    o_ref[...] = jnp.dot(h1, w_ref[...], preferred_element_type=jnp.float32)


@jax.jit
def _tc_mid(p0, p1, b, w):
    return pl.pallas_call(
        _mid_body,
        grid=(N // _ROWS,),
        in_specs=[
            pl.BlockSpec((_ROWS, D), lambda i: (i, 0)),
            pl.BlockSpec((_ROWS, D), lambda i: (i, 0)),
            pl.BlockSpec((1, D), lambda i: (0, 0)),
            pl.BlockSpec((D, D), lambda i: (0, 0)),
        ],
        out_specs=pl.BlockSpec((_ROWS, D), lambda i: (i, 0)),
        out_shape=jax.ShapeDtypeStruct((N, D), jnp.float32),
    )(p0, p1, b, w)


def _final_body(p0_ref, p1_ref, b_ref, o_ref):
    o_ref[...] = jnp.tanh(p0_ref[...] + p1_ref[...] + b_ref[...])


@jax.jit
def _tc_final(p0, p1, b):
    return pl.pallas_call(
        _final_body,
        grid=(N // _ROWS,),
        in_specs=[
            pl.BlockSpec((_ROWS, D), lambda i: (i, 0)),
            pl.BlockSpec((_ROWS, D), lambda i: (i, 0)),
            pl.BlockSpec((1, D), lambda i: (0, 0)),
        ],
        out_specs=pl.BlockSpec((_ROWS, D), lambda i: (i, 0)),
        out_shape=jax.ShapeDtypeStruct((N, D), jnp.float32),
    )(p0, p1, b)


@jax.jit
def kernel(x, edge_index, edge_weight, W1, b1, W2, b2):
    pad = E_PAD - edge_index.shape[1]
    pad_idx = (jnp.arange(pad, dtype=jnp.int32) % N)
    src = jnp.concatenate([edge_index[0].astype(jnp.int32), pad_idx])
    dst = jnp.concatenate([edge_index[1].astype(jnp.int32), pad_idx])
    w = jnp.concatenate([edge_weight.astype(jnp.float32),
                         jnp.zeros((pad,), jnp.float32)])
    b1r = b1.reshape(1, D)
    b2r = b2.reshape(1, D)

    h = _tc_matmul(x, W1)
    p = _sc_aggregate(h, src, dst, w)
    h2 = _tc_mid(p[0, :N], p[1, :N], b1r, W2)
    p2 = _sc_aggregate(h2, src, dst, w)
    return _tc_final(p2[0, :N], p2[1, :N], b2r)


# speedup vs baseline: 1.7697x; 1.0257x over previous
import jax
import jax.numpy as jnp
from jax import lax
from jax.experimental import pallas as pl
from jax.experimental.pallas import tpu as pltpu
from jax.experimental.pallas import tpu_sc as plsc

N = 10000
D = 128
NC = 2
NS = 16
NW = NC * NS
C = 88
NCHUNK = 120
EPW = NCHUNK * C
E_PAD = NW * EPW
N_ACC = 10240
ROWS_PER_TILE = N_ACC // NS
VECS = D // 16


def _sc_aggregate_body(h_hbm, src_hbm, dst_hbm, w_hbm, out_hbm,
                       ed, wf, buf0, buf1, buf2, buf3,
                       esem0, esem1, esem2, esem3, esem4, esem5,
                       rsem0, rsem1, rsem2, rsem3,
                       ssem0, ssem1, ssem2, ssem3, acc):
    c = lax.axis_index("c")
    s = lax.axis_index("s")
    wid = c * NS + s
    ebase = wid * EPW

    esems = (esem0, esem1, esem2, esem3, esem4, esem5)
    bufs = (buf0, buf1, buf2, buf3)
    rsems = (rsem0, rsem1, rsem2, rsem3)
    ssems = (ssem0, ssem1, ssem2, ssem3)


    def _ed_copies(j, es):
        off = ebase + j * C
        return (
            pltpu.make_async_copy(src_hbm.at[pl.ds(off, C)], ed.at[es, 0],
                                  esems[es]),
            pltpu.make_async_copy(dst_hbm.at[pl.ds(off, C)], ed.at[es, 1],
                                  esems[es]),
            pltpu.make_async_copy(w_hbm.at[pl.ds(off, C)],
                                  wf.at[es, pl.ds(0, C)], esems[es]),
        )

    def _ed_issue(j, es):
        for cp in _ed_copies(j, es):
            cp.start()

    def _ed_wait(j, es):
        for cp in _ed_copies(j, es):
            cp.wait()

    def _gather(j, bs, es):
        return pltpu.make_async_copy(h_hbm.at[ed.at[es, 0]], bufs[bs],
                                     rsems[bs])

    def _scatter(bs, es):
        return pltpu.make_async_copy(bufs[bs], acc.at[ed.at[es, 1]],
                                     ssems[bs])

    def _zero_row(e, _):
        for v in range(VECS):
            buf0[e, pl.ds(v * 16, 16)] = jnp.zeros((16,), jnp.float32)
        return _
    lax.fori_loop(0, C, _zero_row, None)
    full = ROWS_PER_TILE // C
    for i in range(full):
        pltpu.sync_copy(buf0, acc.at[pl.ds(s * ROWS_PER_TILE + i * C, C)])
    rem = ROWS_PER_TILE - full * C
    if rem:
        pltpu.sync_copy(buf0.at[pl.ds(0, rem)],
                        acc.at[pl.ds(s * ROWS_PER_TILE + full * C, rem)])

    for j in range(4):
        _ed_issue(j, j)
    _ed_wait(0, 0)
    _gather(0, 0, 0).start()
    _ed_wait(1, 1)
    _gather(1, 1, 1).start()

    plsc.subcore_barrier()

    def _scale(buf, es):
        def _grp(g, _):
            wv = wf[es, pl.ds(g * 16, 16)]
            for k in range(16):
                w = wv[k]
                for v in range(VECS):
                    sl = (g * 16 + k, pl.ds(v * 16, 16))
                    buf[sl] = buf[sl] * w
            return _
        lax.fori_loop(0, C // 16, _grp, None)
        g = C // 16
        for k in range(C - g * 16):
            w = wf[es, pl.ds(g * 16, 16)][k]
            for v in range(VECS):
                sl = (g * 16 + k, pl.ds(v * 16, 16))
                buf[sl] = buf[sl] * w

    def _chunk(j, bs, es):

        @pl.when(j >= 2)
        def _():
            _scatter((bs + 2) % 4, (es + 4) % 6).wait()

        @pl.when(j + 2 < NCHUNK)
        def _():
            _ed_wait(j + 2, (es + 2) % 6)
            _gather(j + 2, (bs + 2) % 4, (es + 2) % 6).start()

        _gather(j, bs, es).wait()
        _scale(bufs[bs], es)
        pltpu.async_copy(bufs[bs], acc.at[ed.at[es, 1]], ssems[bs],
                         add=True)

        @pl.when(j + 4 < NCHUNK)
        def _():
            _ed_issue(j + 4, (es + 4) % 6)

    def _twelve(q, _):
        j0 = q * 12
        for b in range(12):
            _chunk(j0 + b, b % 4, b % 6)
        return _
    lax.fori_loop(0, NCHUNK // 12, _twelve, None)
    _scatter((NCHUNK - 2) % 4, (NCHUNK - 2) % 6).wait()
    _scatter((NCHUNK - 1) % 4, (NCHUNK - 1) % 6).wait()

    plsc.subcore_barrier()
    pltpu.sync_copy(acc.at[pl.ds(s * ROWS_PER_TILE, ROWS_PER_TILE)],
                    out_hbm.at[c, pl.ds(s * ROWS_PER_TILE, ROWS_PER_TILE)])


_sc_aggregate = pl.kernel(
    _sc_aggregate_body,
    out_type=jax.ShapeDtypeStruct((NC, N_ACC, D), jnp.float32),
    mesh=plsc.VectorSubcoreMesh(core_axis_name="c", subcore_axis_name="s"),
    scratch_types=[
        pltpu.VMEM((6, 3, C), jnp.int32),
        pltpu.VMEM((6, 128), jnp.float32),
        pltpu.VMEM((C, D), jnp.float32),
        pltpu.VMEM((C, D), jnp.float32),
        pltpu.VMEM((C, D), jnp.float32),
        pltpu.VMEM((C, D), jnp.float32),
        pltpu.SemaphoreType.DMA,
        pltpu.SemaphoreType.DMA,
        pltpu.SemaphoreType.DMA,
        pltpu.SemaphoreType.DMA,
        pltpu.SemaphoreType.DMA,
        pltpu.SemaphoreType.DMA,
        pltpu.SemaphoreType.DMA,
        pltpu.SemaphoreType.DMA,
        pltpu.SemaphoreType.DMA,
        pltpu.SemaphoreType.DMA,
        pltpu.SemaphoreType.DMA,
        pltpu.SemaphoreType.DMA,
        pltpu.SemaphoreType.DMA,
        pltpu.SemaphoreType.DMA,
        pltpu.VMEM_SHARED((N_ACC, D), jnp.float32),
    ],
)


_ROWS = 2000


def _mid_body(p0_ref, p1_ref, b_ref, w1_ref, w2_ref, o_ref):
    t = jnp.dot(p0_ref[...] + p1_ref[...], w1_ref[...],
                preferred_element_type=jnp.float32) + b_ref[...]
    o_ref[...] = jnp.dot(jnp.tanh(t), w2_ref[...],
                         preferred_element_type=jnp.float32)


@jax.jit
def _tc_mid(p0, p1, b, w1, w2):
    return pl.pallas_call(
        _mid_body,
        grid=(N // _ROWS,),
        in_specs=[
            pl.BlockSpec((_ROWS, D), lambda i: (i, 0)),
            pl.BlockSpec((_ROWS, D), lambda i: (i, 0)),
            pl.BlockSpec((1, D), lambda i: (0, 0)),
            pl.BlockSpec((D, D), lambda i: (0, 0)),
            pl.BlockSpec((D, D), lambda i: (0, 0)),
        ],
        out_specs=pl.BlockSpec((_ROWS, D), lambda i: (i, 0)),
        out_shape=jax.ShapeDtypeStruct((N, D), jnp.float32),
    )(p0, p1, b, w1, w2)


def _final_body(p0_ref, p1_ref, b_ref, o_ref):
    o_ref[...] = jnp.tanh(p0_ref[...] + p1_ref[...] + b_ref[...])


@jax.jit
def _tc_final(p0, p1, b):
    return pl.pallas_call(
        _final_body,
        grid=(N // _ROWS,),
        in_specs=[
            pl.BlockSpec((_ROWS, D), lambda i: (i, 0)),
            pl.BlockSpec((_ROWS, D), lambda i: (i, 0)),
            pl.BlockSpec((1, D), lambda i: (0, 0)),
        ],
        out_specs=pl.BlockSpec((_ROWS, D), lambda i: (i, 0)),
        out_shape=jax.ShapeDtypeStruct((N, D), jnp.float32),
    )(p0, p1, b)


@jax.jit
def kernel(x, edge_index, edge_weight, W1, b1, W2, b2):
    pad = E_PAD - edge_index.shape[1]
    pad_idx = (jnp.arange(pad, dtype=jnp.int32) % N)
    src = jnp.concatenate([edge_index[0].astype(jnp.int32), pad_idx])
    dst = jnp.concatenate([edge_index[1].astype(jnp.int32), pad_idx])
    w = jnp.concatenate([edge_weight.astype(jnp.float32),
                         jnp.zeros((pad,), jnp.float32)])
    b1r = b1.reshape(1, D)
    b2r = b2.reshape(1, D)

    p = _sc_aggregate(x, src, dst, w)
    h2 = _tc_mid(p[0, :N], p[1, :N], b1r, W1, W2)
    p2 = _sc_aggregate(h2, src, dst, w)
    return _tc_final(p2[0, :N], p2[1, :N], b2r)
